# Initial kernel scaffold; baseline (speedup 1.0000x reference)
import jax
import jax.numpy as jnp
from jax.experimental import pallas as pl


def kernel(x, frag_idx, W1, b1, Wout, bout):
    raise NotImplementedError("write your pallas kernel here")


# speedup vs baseline: 1.5710x; 1.5710x over previous
import functools

import jax
import jax.numpy as jnp
from jax import lax
from jax.experimental import pallas as pl
from jax.experimental.pallas import tpu as pltpu
from jax.experimental.pallas import tpu_sc as plsc

N_ATOMS = 100000
D_FEAT = 128
MID = 32
OUT_DIM = 3
N_FRAG = 200000

NC = 2
NS = 16
NW = NC * NS
C = 128
PAD_TOTAL = -(-N_FRAG // (NW * C * 2)) * (NW * C * 2)
W_PER = PAD_TOTAL // NW
NCH = W_PER // C
NG = C // 16

ROW_BLK = 2000

_GDN = lax.GatherDimensionNumbers(
    offset_dims=(), collapsed_slice_dims=(0,), start_index_map=(0,))


def _lane_pick(vec, idx16):
    return lax.gather(vec, idx16[:, None], _GDN, (1,),
                      mode=lax.GatherScatterMode.PROMISE_IN_BOUNDS)


def _project_body(x_ref, wu_ref, wb_ref, bu_ref, bb_ref, u_ref, tb_ref):
    xb = x_ref[...]
    u_ref[...] = (
        jnp.dot(xb, wu_ref[...], preferred_element_type=jnp.float32,
                precision=lax.Precision.HIGHEST)
        + bu_ref[...]
    )
    tb_ref[...] = (
        jnp.dot(xb, wb_ref[...], preferred_element_type=jnp.float32,
                precision=lax.Precision.HIGHEST)
        + bb_ref[...]
    )


def _project(x, wu, wb, bu, bb):
    n = x.shape[0]
    grid = n // ROW_BLK
    return pl.pallas_call(
        _project_body,
        grid=(grid,),
        in_specs=[
            pl.BlockSpec((ROW_BLK, D_FEAT), lambda i: (i, 0)),
            pl.BlockSpec((D_FEAT, 2 * MID), lambda i: (0, 0)),
            pl.BlockSpec((D_FEAT, MID), lambda i: (0, 0)),
            pl.BlockSpec((1, 2 * MID), lambda i: (0, 0)),
            pl.BlockSpec((1, MID), lambda i: (0, 0)),
        ],
        out_specs=[
            pl.BlockSpec((ROW_BLK, 2 * MID), lambda i: (i, 0)),
            pl.BlockSpec((ROW_BLK, MID), lambda i: (i, 0)),
        ],
        out_shape=[
            jax.ShapeDtypeStruct((n, 2 * MID), jnp.float32),
            jax.ShapeDtypeStruct((n, MID), jnp.float32),
        ],
    )(x, wu, wb, bu, bb)


def _sc_readout_body(u_hbm, tb_hbm, fi0, fi1, fi2, woutf, boutp, out_hbm,
                     idx_a, idx_b, r0_a, r0_b, r2_a, r2_b, rb_a, rb_b,
                     out_a, out_b, wout_v, bout_v, sem0, sem1):
    wid = lax.axis_index("s") * NC + lax.axis_index("c")
    wbase = wid * W_PER
    sems = (sem0, sem1)
    idxs = (idx_a, idx_b)
    r0s = (r0_a, r0_b)
    r2s = (r2_a, r2_b)
    rbs = (rb_a, rb_b)
    outs = (out_a, out_b)

    pltpu.sync_copy(woutf, wout_v)
    pltpu.sync_copy(boutp, bout_v)

    fis = (fi0, fi1, fi2)

    def stage(c, b):
        base = wbase + c * C
        for r in range(3):
            pltpu.sync_copy(fis[r].at[pl.ds(base, C)], idxs[b].at[r])
        pltpu.async_copy(u_hbm.at[idxs[b].at[0]],
                         r0s[b], sems[b])
        pltpu.async_copy(tb_hbm.at[idxs[b].at[1]],
                         rbs[b], sems[b])
        pltpu.async_copy(u_hbm.at[idxs[b].at[2]],
                         r2s[b], sems[b])

    def wait(b):
        pltpu.make_async_copy(u_hbm.at[idxs[b].at[0]],
                              r0s[b], sems[b]).wait()
        pltpu.make_async_copy(tb_hbm.at[idxs[b].at[1]],
                              rbs[b], sems[b]).wait()
        pltpu.make_async_copy(u_hbm.at[idxs[b].at[2]],
                              r2s[b], sems[b]).wait()

    bv = bout_v[...]
    w_lo = [wout_v[j, pl.ds(0, 16)] for j in range(OUT_DIM)]
    w_hi = [wout_v[j, pl.ds(16, 16)] for j in range(OUT_DIM)]

    def compute(c, b):
        r0 = r0s[b]
        r2 = r2s[b]
        rb = rbs[b]
        ov = outs[b]

        @pl.loop(0, NG)
        def _(g):
            rows = lax.iota(jnp.int32, 16) + g * 16
            acc = [_lane_pick(bv, jnp.full((16,), j, jnp.int32))
                   for j in range(OUT_DIM)]
            for k in range(MID):
                ck = jnp.full((16,), k, jnp.int32)
                ck2 = jnp.full((16,), MID + k, jnp.int32)
                a0 = plsc.load_gather(r0, [rows, ck])
                c0 = plsc.load_gather(r0, [rows, ck2])
                a2 = plsc.load_gather(r2, [rows, ck])
                c2 = plsc.load_gather(r2, [rows, ck2])
                t1 = plsc.load_gather(rb, [rows, ck])
                fwd = a0 + c2 + t1
                bwd = a2 + c0 + t1
                p = jnp.maximum(fwd, 0.0) + jnp.maximum(bwd, 0.0)
                kv = jnp.full((16,), k % 16, jnp.int32)
                for j in range(OUT_DIM):
                    wsrc = w_lo[j] if k < 16 else w_hi[j]
                    acc[j] = acc[j] + p * _lane_pick(wsrc, kv)
            for j in range(OUT_DIM):
                plsc.store_scatter(ov, [rows, jnp.full((16,), j, jnp.int32)],
                                   acc[j])

        pltpu.sync_copy(outs[b], out_hbm.at[pl.ds(wbase + c * C, C)])

    stage(0, 0)

    @pl.loop(0, NCH // 2)
    def _(i):
        c0 = i * 2
        stage(c0 + 1, 1)
        wait(0)
        compute(c0, 0)

        @pl.when(i < NCH // 2 - 1)
        def _():
            stage(c0 + 2, 0)

        wait(1)
        compute(c0 + 1, 1)


_sc_readout = functools.partial(
    pl.kernel,
    out_type=jax.ShapeDtypeStruct((PAD_TOTAL, OUT_DIM), jnp.float32),
    mesh=plsc.VectorSubcoreMesh(core_axis_name="c", subcore_axis_name="s",
                                num_cores=NC, num_subcores=NS),
    compiler_params=pltpu.CompilerParams(needs_layout_passes=False,
                                         use_tc_tiling_on_sc=False),
    scratch_types=[
        pltpu.VMEM((3, C), jnp.int32),
        pltpu.VMEM((3, C), jnp.int32),
        pltpu.VMEM((C, 2 * MID), jnp.float32),
        pltpu.VMEM((C, 2 * MID), jnp.float32),
        pltpu.VMEM((C, 2 * MID), jnp.float32),
        pltpu.VMEM((C, 2 * MID), jnp.float32),
        pltpu.VMEM((C, MID), jnp.float32),
        pltpu.VMEM((C, MID), jnp.float32),
        pltpu.VMEM((C, OUT_DIM), jnp.float32),
        pltpu.VMEM((C, OUT_DIM), jnp.float32),
        pltpu.VMEM((OUT_DIM, MID), jnp.float32),
        pltpu.VMEM((16,), jnp.float32),
        pltpu.SemaphoreType.DMA,
        pltpu.SemaphoreType.DMA,
    ],
)(_sc_readout_body)


def kernel(x, frag_idx, W1, b1, Wout, bout):
    d = x.shape[1]
    wu = jnp.concatenate([W1[0:d], W1[2 * d:3 * d]], axis=1)
    wb = W1[d:2 * d]
    third = 1.0 / 3.0
    bu = (jnp.concatenate([b1, b1]) * third)[None, :]
    bb = (b1 * third)[None, :]
    u_tab, tb_tab = _project(x, wu, wb, bu, bb)

    fi = jnp.pad(frag_idx, ((0, 0), (0, PAD_TOTAL - frag_idx.shape[1])))
    woutf = Wout.T
    boutp = jnp.pad(bout, (0, 16 - bout.shape[0]))
    out = _sc_readout(u_tab, tb_tab, fi[0], fi[1], fi[2], woutf, boutp)
    return out[:frag_idx.shape[1]]


# speedup vs baseline: 1.6751x; 1.0662x over previous
import functools

import jax
import jax.numpy as jnp
from jax import lax
from jax.experimental import pallas as pl
from jax.experimental.pallas import tpu as pltpu
from jax.experimental.pallas import tpu_sc as plsc

N_ATOMS = 100000
D_FEAT = 128
MID = 32
OUT_DIM = 3
N_FRAG = 200000

NC = 2
NS = 16
NW = NC * NS
C = 320
PAD_TOTAL = -(-N_FRAG // (NW * C * 2)) * (NW * C * 2)
W_PER = PAD_TOTAL // NW
NCH = W_PER // C
NG = C // 16

ROW_BLK = 2000

_GDN = lax.GatherDimensionNumbers(
    offset_dims=(), collapsed_slice_dims=(0,), start_index_map=(0,))


def _lane_pick(vec, idx16):
    return lax.gather(vec, idx16[:, None], _GDN, (1,),
                      mode=lax.GatherScatterMode.PROMISE_IN_BOUNDS)


def _project_body(x_ref, wu_ref, wb_ref, bu_ref, bb_ref, u_ref, tb_ref):
    xb = x_ref[...]
    u_ref[...] = (
        jnp.dot(xb, wu_ref[...], preferred_element_type=jnp.float32,
                precision=lax.Precision.HIGHEST)
        + bu_ref[...]
    )
    tb_ref[...] = (
        jnp.dot(xb, wb_ref[...], preferred_element_type=jnp.float32,
                precision=lax.Precision.HIGHEST)
        + bb_ref[...]
    )


def _project(x, wu, wb, bu, bb):
    n = x.shape[0]
    grid = n // ROW_BLK
    return pl.pallas_call(
        _project_body,
        grid=(grid,),
        in_specs=[
            pl.BlockSpec((ROW_BLK, D_FEAT), lambda i: (i, 0)),
            pl.BlockSpec((D_FEAT, 2 * MID), lambda i: (0, 0)),
            pl.BlockSpec((D_FEAT, MID), lambda i: (0, 0)),
            pl.BlockSpec((1, 2 * MID), lambda i: (0, 0)),
            pl.BlockSpec((1, MID), lambda i: (0, 0)),
        ],
        out_specs=[
            pl.BlockSpec((ROW_BLK, 2 * MID), lambda i: (i, 0)),
            pl.BlockSpec((ROW_BLK, MID), lambda i: (i, 0)),
        ],
        out_shape=[
            jax.ShapeDtypeStruct((n, 2 * MID), jnp.float32),
            jax.ShapeDtypeStruct((n, MID), jnp.float32),
        ],
    )(x, wu, wb, bu, bb)


def _sc_readout_body(u_hbm, tb_hbm, fi0, fi1, fi2, woutf, boutp, out_hbm,
                     idx_all, r0_a, r0_b, r2_a, r2_b, rb_a, rb_b,
                     out_a, out_b, wout_v, bout_v, sem0, sem1):
    wid = lax.axis_index("s") * NC + lax.axis_index("c")
    wbase = wid * W_PER
    sems = (sem0, sem1)
    r0s = (r0_a, r0_b)
    r2s = (r2_a, r2_b)
    rbs = (rb_a, rb_b)
    outs = (out_a, out_b)

    pltpu.sync_copy(woutf, wout_v)
    pltpu.sync_copy(boutp, bout_v)

    pltpu.sync_copy(fi0.at[pl.ds(wbase, W_PER)], idx_all.at[0])
    pltpu.sync_copy(fi1.at[pl.ds(wbase, W_PER)], idx_all.at[1])
    pltpu.sync_copy(fi2.at[pl.ds(wbase, W_PER)], idx_all.at[2])

    def stage(c, b):
        off = c * C
        pltpu.async_copy(u_hbm.at[idx_all.at[0, pl.ds(off, C)]],
                         r0s[b], sems[b])
        pltpu.async_copy(tb_hbm.at[idx_all.at[1, pl.ds(off, C)]],
                         rbs[b], sems[b])
        pltpu.async_copy(u_hbm.at[idx_all.at[2, pl.ds(off, C)]],
                         r2s[b], sems[b])

    def wait(c, b):
        off = c * C
        pltpu.make_async_copy(u_hbm.at[idx_all.at[0, pl.ds(off, C)]],
                              r0s[b], sems[b]).wait()
        pltpu.make_async_copy(tb_hbm.at[idx_all.at[1, pl.ds(off, C)]],
                              rbs[b], sems[b]).wait()
        pltpu.make_async_copy(u_hbm.at[idx_all.at[2, pl.ds(off, C)]],
                              r2s[b], sems[b]).wait()

    bv = bout_v[...]
    w_lo = [wout_v[j, pl.ds(0, 16)] for j in range(OUT_DIM)]
    w_hi = [wout_v[j, pl.ds(16, 16)] for j in range(OUT_DIM)]

    def compute(c, b):
        r0 = r0s[b]
        r2 = r2s[b]
        rb = rbs[b]
        ov = outs[b]

        @pl.loop(0, NG)
        def _(g):
            rows = lax.iota(jnp.int32, 16) + g * 16
            acc = [_lane_pick(bv, jnp.full((16,), j, jnp.int32))
                   for j in range(OUT_DIM)]
            for k in range(MID):
                ck = jnp.full((16,), k, jnp.int32)
                ck2 = jnp.full((16,), MID + k, jnp.int32)
                a0 = plsc.load_gather(r0, [rows, ck])
                c0 = plsc.load_gather(r0, [rows, ck2])
                a2 = plsc.load_gather(r2, [rows, ck])
                c2 = plsc.load_gather(r2, [rows, ck2])
                t1 = plsc.load_gather(rb, [rows, ck])
                fwd = a0 + c2 + t1
                bwd = a2 + c0 + t1
                p = jnp.maximum(fwd, 0.0) + jnp.maximum(bwd, 0.0)
                kv = jnp.full((16,), k % 16, jnp.int32)
                for j in range(OUT_DIM):
                    wsrc = w_lo[j] if k < 16 else w_hi[j]
                    acc[j] = acc[j] + p * _lane_pick(wsrc, kv)
            for j in range(OUT_DIM):
                plsc.store_scatter(ov, [rows, jnp.full((16,), j, jnp.int32)],
                                   acc[j])

        pltpu.sync_copy(outs[b], out_hbm.at[pl.ds(wbase + c * C, C)])

    stage(0, 0)

    @pl.loop(0, NCH // 2)
    def _(i):
        c0 = i * 2
        stage(c0 + 1, 1)
        wait(c0, 0)
        compute(c0, 0)

        @pl.when(i < NCH // 2 - 1)
        def _():
            stage(c0 + 2, 0)

        wait(c0 + 1, 1)
        compute(c0 + 1, 1)


_sc_readout = functools.partial(
    pl.kernel,
    out_type=jax.ShapeDtypeStruct((PAD_TOTAL, OUT_DIM), jnp.float32),
    mesh=plsc.VectorSubcoreMesh(core_axis_name="c", subcore_axis_name="s",
                                num_cores=NC, num_subcores=NS),
    compiler_params=pltpu.CompilerParams(needs_layout_passes=False,
                                         use_tc_tiling_on_sc=False),
    scratch_types=[
        pltpu.VMEM((3, W_PER), jnp.int32),
        pltpu.VMEM((C, 2 * MID), jnp.float32),
        pltpu.VMEM((C, 2 * MID), jnp.float32),
        pltpu.VMEM((C, 2 * MID), jnp.float32),
        pltpu.VMEM((C, 2 * MID), jnp.float32),
        pltpu.VMEM((C, MID), jnp.float32),
        pltpu.VMEM((C, MID), jnp.float32),
        pltpu.VMEM((C, OUT_DIM), jnp.float32),
        pltpu.VMEM((C, OUT_DIM), jnp.float32),
        pltpu.VMEM((OUT_DIM, MID), jnp.float32),
        pltpu.VMEM((16,), jnp.float32),
        pltpu.SemaphoreType.DMA,
        pltpu.SemaphoreType.DMA,
    ],
)(_sc_readout_body)


def kernel(x, frag_idx, W1, b1, Wout, bout):
    d = x.shape[1]
    wu = jnp.concatenate([W1[0:d], W1[2 * d:3 * d]], axis=1)
    wb = W1[d:2 * d]
    third = 1.0 / 3.0
    bu = (jnp.concatenate([b1, b1]) * third)[None, :]
    bb = (b1 * third)[None, :]
    u_tab, tb_tab = _project(x, wu, wb, bu, bb)

    fi = jnp.pad(frag_idx, ((0, 0), (0, PAD_TOTAL - frag_idx.shape[1])))
    woutf = Wout.T
    boutp = jnp.pad(bout, (0, 16 - bout.shape[0]))
    out = _sc_readout(u_tab, tb_tab, fi[0], fi[1], fi[2], woutf, boutp)
    return out[:frag_idx.shape[1]]


# speedup vs baseline: 2.2225x; 1.3268x over previous
import functools

import jax
import jax.numpy as jnp
from jax import lax
from jax.experimental import pallas as pl
from jax.experimental.pallas import tpu as pltpu
from jax.experimental.pallas import tpu_sc as plsc

N_ATOMS = 100000
D_FEAT = 128
MID = 32
OUT_DIM = 3
N_FRAG = 200000

NC = 2
NS = 16
NW = NC * NS
C = 320
PAD_TOTAL = -(-N_FRAG // (NW * C * 2)) * (NW * C * 2)
W_PER = PAD_TOTAL // NW
NCH = W_PER // C
NG = C // 16

ROW_BLK = 2000

_GDN = lax.GatherDimensionNumbers(
    offset_dims=(), collapsed_slice_dims=(0,), start_index_map=(0,))


def _lane_pick(vec, idx16):
    return lax.gather(vec, idx16[:, None], _GDN, (1,),
                      mode=lax.GatherScatterMode.PROMISE_IN_BOUNDS)


def _project_body(x_ref, wu_ref, wb_ref, bu_ref, bb_ref, u_ref, tb_ref):
    xb = x_ref[...]
    u_ref[...] = (
        jnp.dot(xb, wu_ref[...], preferred_element_type=jnp.float32,
                precision=lax.Precision.DEFAULT)
        + bu_ref[...]
    )
    tb_ref[...] = (
        jnp.dot(xb, wb_ref[...], preferred_element_type=jnp.float32,
                precision=lax.Precision.DEFAULT)
        + bb_ref[...]
    )


def _project(x, wu, wb, bu, bb):
    n = x.shape[0]
    grid = n // ROW_BLK
    return pl.pallas_call(
        _project_body,
        grid=(grid,),
        in_specs=[
            pl.BlockSpec((ROW_BLK, D_FEAT), lambda i: (i, 0)),
            pl.BlockSpec((D_FEAT, 2 * MID), lambda i: (0, 0)),
            pl.BlockSpec((D_FEAT, MID), lambda i: (0, 0)),
            pl.BlockSpec((1, 2 * MID), lambda i: (0, 0)),
            pl.BlockSpec((1, MID), lambda i: (0, 0)),
        ],
        out_specs=[
            pl.BlockSpec((ROW_BLK, 2 * MID), lambda i: (i, 0)),
            pl.BlockSpec((ROW_BLK, MID), lambda i: (i, 0)),
        ],
        out_shape=[
            jax.ShapeDtypeStruct((n, 2 * MID), jnp.float32),
            jax.ShapeDtypeStruct((n, MID), jnp.float32),
        ],
    )(x, wu, wb, bu, bb)


def _sc_readout_body(u_hbm, tb_hbm, fi0, fi1, fi2, woutf, boutp, out_hbm,
                     idx_all, r0_a, r0_b, r2_a, r2_b, rb_a, rb_b,
                     out_a, out_b, wout_v, bout_v, sem0, sem1):
    wid = lax.axis_index("s") * NC + lax.axis_index("c")
    wbase = wid * W_PER
    sems = (sem0, sem1)
    r0s = (r0_a, r0_b)
    r2s = (r2_a, r2_b)
    rbs = (rb_a, rb_b)
    outs = (out_a, out_b)

    pltpu.sync_copy(woutf, wout_v)
    pltpu.sync_copy(boutp, bout_v)

    pltpu.sync_copy(fi0.at[pl.ds(wbase, W_PER)], idx_all.at[0])
    pltpu.sync_copy(fi1.at[pl.ds(wbase, W_PER)], idx_all.at[1])
    pltpu.sync_copy(fi2.at[pl.ds(wbase, W_PER)], idx_all.at[2])

    def stage(c, b):
        off = c * C
        pltpu.async_copy(u_hbm.at[idx_all.at[0, pl.ds(off, C)]],
                         r0s[b], sems[b])
        pltpu.async_copy(tb_hbm.at[idx_all.at[1, pl.ds(off, C)]],
                         rbs[b], sems[b])
        pltpu.async_copy(u_hbm.at[idx_all.at[2, pl.ds(off, C)]],
                         r2s[b], sems[b])

    def wait(c, b):
        off = c * C
        pltpu.make_async_copy(u_hbm.at[idx_all.at[0, pl.ds(off, C)]],
                              r0s[b], sems[b]).wait()
        pltpu.make_async_copy(tb_hbm.at[idx_all.at[1, pl.ds(off, C)]],
                              rbs[b], sems[b]).wait()
        pltpu.make_async_copy(u_hbm.at[idx_all.at[2, pl.ds(off, C)]],
                              r2s[b], sems[b]).wait()

    bv = bout_v[...]
    w_lo = [wout_v[pl.ds(j * MID, 16)] for j in range(OUT_DIM)]
    w_hi = [wout_v[pl.ds(j * MID + 16, 16)] for j in range(OUT_DIM)]
    bj = [_lane_pick(bv, jnp.full((16,), j, jnp.int32)) for j in range(OUT_DIM)]
    lane_last = lax.iota(jnp.int32, 16) == 15

    def compute(c, b):
        r0 = r0s[b]
        r2 = r2s[b]
        rb = rbs[b]
        ov = outs[b]

        @pl.loop(0, C, unroll=4)
        def _(f):
            u0a = r0[f, pl.ds(0, 16)]
            u0b = r0[f, pl.ds(16, 16)]
            u0c = r0[f, pl.ds(32, 16)]
            u0d = r0[f, pl.ds(48, 16)]
            u2a = r2[f, pl.ds(0, 16)]
            u2b = r2[f, pl.ds(16, 16)]
            u2c = r2[f, pl.ds(32, 16)]
            u2d = r2[f, pl.ds(48, 16)]
            tl = rb[f, pl.ds(0, 16)]
            th = rb[f, pl.ds(16, 16)]
            zero = jnp.float32(0.0)
            p_lo = (jnp.maximum((u0a + u2c) + tl, zero)
                    + jnp.maximum((u2a + u0c) + tl, zero))
            p_hi = (jnp.maximum((u0b + u2d) + th, zero)
                    + jnp.maximum((u2b + u0d) + th, zero))
            for j in range(OUT_DIM):
                t = p_lo * w_lo[j] + p_hi * w_hi[j]
                s = plsc.cumsum(t) + bj[j]
                plsc.store_scatter(ov, [jnp.full((16,), f * OUT_DIM + j,
                                                 jnp.int32)],
                                   s, mask=lane_last)

        pltpu.sync_copy(ov, out_hbm.at[pl.ds((wbase + c * C) * OUT_DIM,
                                             C * OUT_DIM)])

    stage(0, 0)

    @pl.loop(0, NCH // 2)
    def _(i):
        c0 = i * 2
        stage(c0 + 1, 1)
        wait(c0, 0)
        compute(c0, 0)

        @pl.when(i < NCH // 2 - 1)
        def _():
            stage(c0 + 2, 0)

        wait(c0 + 1, 1)
        compute(c0 + 1, 1)


_sc_readout = functools.partial(
    pl.kernel,
    out_type=jax.ShapeDtypeStruct((PAD_TOTAL * OUT_DIM,), jnp.float32),
    mesh=plsc.VectorSubcoreMesh(core_axis_name="c", subcore_axis_name="s",
                                num_cores=NC, num_subcores=NS),
    compiler_params=pltpu.CompilerParams(needs_layout_passes=False,
                                         use_tc_tiling_on_sc=False),
    scratch_types=[
        pltpu.VMEM((3, W_PER), jnp.int32),
        pltpu.VMEM((C, 2 * MID), jnp.float32),
        pltpu.VMEM((C, 2 * MID), jnp.float32),
        pltpu.VMEM((C, 2 * MID), jnp.float32),
        pltpu.VMEM((C, 2 * MID), jnp.float32),
        pltpu.VMEM((C, MID), jnp.float32),
        pltpu.VMEM((C, MID), jnp.float32),
        pltpu.VMEM((C * OUT_DIM,), jnp.float32),
        pltpu.VMEM((C * OUT_DIM,), jnp.float32),
        pltpu.VMEM((OUT_DIM * MID,), jnp.float32),
        pltpu.VMEM((16,), jnp.float32),
        pltpu.SemaphoreType.DMA,
        pltpu.SemaphoreType.DMA,
    ],
)(_sc_readout_body)


def kernel(x, frag_idx, W1, b1, Wout, bout):
    d = x.shape[1]
    wu = jnp.concatenate([W1[0:d], W1[2 * d:3 * d]], axis=1)
    wb = W1[d:2 * d]
    third = 1.0 / 3.0
    bu = (jnp.concatenate([b1, b1]) * third)[None, :]
    bb = (b1 * third)[None, :]
    u_tab, tb_tab = _project(x, wu, wb, bu, bb)

    fi = jnp.pad(frag_idx, ((0, 0), (0, PAD_TOTAL - frag_idx.shape[1])))
    woutf = Wout.T.reshape(-1)
    boutp = jnp.pad(bout, (0, 16 - bout.shape[0]))
    out = _sc_readout(u_tab, tb_tab, fi[0], fi[1], fi[2], woutf, boutp)
    return out.reshape(PAD_TOTAL, OUT_DIM)[:frag_idx.shape[1]]


# speedup vs baseline: 2.8535x; 1.2840x over previous
import functools

import jax
import jax.numpy as jnp
from jax import lax
from jax.experimental import pallas as pl
from jax.experimental.pallas import tpu as pltpu
from jax.experimental.pallas import tpu_sc as plsc

N_ATOMS = 100000
D_FEAT = 128
MID = 32
OUT_DIM = 3
N_FRAG = 200000

NC = 2
NS = 16
NW = NC * NS
C = 320
PAD_TOTAL = -(-N_FRAG // (NW * C * 2)) * (NW * C * 2)
W_PER = PAD_TOTAL // NW
NCH = W_PER // C
NG = C // 16

ROW_BLK = 2000

_GDN = lax.GatherDimensionNumbers(
    offset_dims=(), collapsed_slice_dims=(0,), start_index_map=(0,))


def _lane_pick(vec, idx16):
    return lax.gather(vec, idx16[:, None], _GDN, (1,),
                      mode=lax.GatherScatterMode.PROMISE_IN_BOUNDS)


def _rnd_bf16_bits(x):
    b = lax.bitcast_convert_type(x, jnp.int32)
    return b + jnp.int32(0x7FFF) + ((b >> 16) & jnp.int32(1))


def _project_body(x_ref, wu_ref, wb_ref, bu_ref, bb_ref, u_ref, tb_ref):
    xb = x_ref[...]
    u = (jnp.dot(xb, wu_ref[...], preferred_element_type=jnp.float32,
                 precision=lax.Precision.DEFAULT) + bu_ref[...])
    tb = (jnp.dot(xb, wb_ref[...], preferred_element_type=jnp.float32,
                  precision=lax.Precision.DEFAULT) + bb_ref[...])
    hi_mask = jnp.int32(-65536)
    ta_b = _rnd_bf16_bits(u[:, :MID])
    tc_b = _rnd_bf16_bits(u[:, MID:])
    u_ref[...] = ((ta_b >> 16) & jnp.int32(0xFFFF)) | (tc_b & hi_mask)
    tbl_b = _rnd_bf16_bits(tb[:, :16])
    tbh_b = _rnd_bf16_bits(tb[:, 16:])
    tb_ref[...] = ((tbl_b >> 16) & jnp.int32(0xFFFF)) | (tbh_b & hi_mask)


def _project(x, wu, wb, bu, bb):
    n = x.shape[0]
    grid = n // ROW_BLK
    return pl.pallas_call(
        _project_body,
        grid=(grid,),
        in_specs=[
            pl.BlockSpec((ROW_BLK, D_FEAT), lambda i: (i, 0)),
            pl.BlockSpec((D_FEAT, 2 * MID), lambda i: (0, 0)),
            pl.BlockSpec((D_FEAT, MID), lambda i: (0, 0)),
            pl.BlockSpec((1, 2 * MID), lambda i: (0, 0)),
            pl.BlockSpec((1, MID), lambda i: (0, 0)),
        ],
        out_specs=[
            pl.BlockSpec((ROW_BLK, MID), lambda i: (i, 0)),
            pl.BlockSpec((ROW_BLK, MID // 2), lambda i: (i, 0)),
        ],
        out_shape=[
            jax.ShapeDtypeStruct((n, MID), jnp.int32),
            jax.ShapeDtypeStruct((n, MID // 2), jnp.int32),
        ],
    )(x, wu, wb, bu, bb)


def _sc_readout_body(u_hbm, tb_hbm, fi0, fi1, fi2, woutf, boutp, out_hbm,
                     idx_all, r0_a, r0_b, r2_a, r2_b, rb_a, rb_b,
                     out_a, out_b, wout_v, bout_v, sem0, sem1):
    wid = lax.axis_index("s") * NC + lax.axis_index("c")
    wbase = wid * W_PER
    sems = (sem0, sem1)
    r0s = (r0_a, r0_b)
    r2s = (r2_a, r2_b)
    rbs = (rb_a, rb_b)
    outs = (out_a, out_b)

    pltpu.sync_copy(woutf, wout_v)
    pltpu.sync_copy(boutp, bout_v)

    pltpu.sync_copy(fi0.at[pl.ds(wbase, W_PER)], idx_all.at[0])
    pltpu.sync_copy(fi1.at[pl.ds(wbase, W_PER)], idx_all.at[1])
    pltpu.sync_copy(fi2.at[pl.ds(wbase, W_PER)], idx_all.at[2])

    def stage(c, b):
        off = c * C
        pltpu.async_copy(u_hbm.at[idx_all.at[0, pl.ds(off, C)]],
                         r0s[b], sems[b])
        pltpu.async_copy(tb_hbm.at[idx_all.at[1, pl.ds(off, C)]],
                         rbs[b], sems[b])
        pltpu.async_copy(u_hbm.at[idx_all.at[2, pl.ds(off, C)]],
                         r2s[b], sems[b])

    def wait(c, b):
        off = c * C
        pltpu.make_async_copy(u_hbm.at[idx_all.at[0, pl.ds(off, C)]],
                              r0s[b], sems[b]).wait()
        pltpu.make_async_copy(tb_hbm.at[idx_all.at[1, pl.ds(off, C)]],
                              rbs[b], sems[b]).wait()
        pltpu.make_async_copy(u_hbm.at[idx_all.at[2, pl.ds(off, C)]],
                              r2s[b], sems[b]).wait()

    bv = bout_v[...]
    w_lo = [wout_v[pl.ds(j * MID, 16)] for j in range(OUT_DIM)]
    w_hi = [wout_v[pl.ds(j * MID + 16, 16)] for j in range(OUT_DIM)]
    bj = [_lane_pick(bv, jnp.full((16,), j, jnp.int32)) for j in range(OUT_DIM)]
    lane_last = lax.iota(jnp.int32, 16) == 15

    def compute(c, b):
        r0 = r0s[b]
        r2 = r2s[b]
        rb = rbs[b]
        ov = outs[b]

        @pl.loop(0, C, unroll=4)
        def _(f):
            hi_m = jnp.int32(-65536)
            p0l = r0[f, pl.ds(0, 16)]
            p0h = r0[f, pl.ds(16, 16)]
            p2l = r2[f, pl.ds(0, 16)]
            p2h = r2[f, pl.ds(16, 16)]
            ptb = rb[f, pl.ds(0, 16)]
            u0a = plsc.bitcast(p0l << 16, jnp.float32)
            u0c = plsc.bitcast(p0l & hi_m, jnp.float32)
            u0b = plsc.bitcast(p0h << 16, jnp.float32)
            u0d = plsc.bitcast(p0h & hi_m, jnp.float32)
            u2a = plsc.bitcast(p2l << 16, jnp.float32)
            u2c = plsc.bitcast(p2l & hi_m, jnp.float32)
            u2b = plsc.bitcast(p2h << 16, jnp.float32)
            u2d = plsc.bitcast(p2h & hi_m, jnp.float32)
            tl = plsc.bitcast(ptb << 16, jnp.float32)
            th = plsc.bitcast(ptb & hi_m, jnp.float32)
            zero = jnp.float32(0.0)
            p_lo = (jnp.maximum((u0a + u2c) + tl, zero)
                    + jnp.maximum((u2a + u0c) + tl, zero))
            p_hi = (jnp.maximum((u0b + u2d) + th, zero)
                    + jnp.maximum((u2b + u0d) + th, zero))
            for j in range(OUT_DIM):
                t = p_lo * w_lo[j] + p_hi * w_hi[j]
                s = plsc.cumsum(t) + bj[j]
                plsc.store_scatter(ov, [jnp.full((16,), f * OUT_DIM + j,
                                                 jnp.int32)],
                                   s, mask=lane_last)

        pltpu.sync_copy(ov, out_hbm.at[pl.ds((wbase + c * C) * OUT_DIM,
                                             C * OUT_DIM)])

    stage(0, 0)

    @pl.loop(0, NCH // 2)
    def _(i):
        c0 = i * 2
        stage(c0 + 1, 1)
        wait(c0, 0)
        compute(c0, 0)

        @pl.when(i < NCH // 2 - 1)
        def _():
            stage(c0 + 2, 0)

        wait(c0 + 1, 1)
        compute(c0 + 1, 1)


_sc_readout = functools.partial(
    pl.kernel,
    out_type=jax.ShapeDtypeStruct((PAD_TOTAL * OUT_DIM,), jnp.float32),
    mesh=plsc.VectorSubcoreMesh(core_axis_name="c", subcore_axis_name="s",
                                num_cores=NC, num_subcores=NS),
    compiler_params=pltpu.CompilerParams(needs_layout_passes=False,
                                         use_tc_tiling_on_sc=False),
    scratch_types=[
        pltpu.VMEM((3, W_PER), jnp.int32),
        pltpu.VMEM((C, MID), jnp.int32),
        pltpu.VMEM((C, MID), jnp.int32),
        pltpu.VMEM((C, MID), jnp.int32),
        pltpu.VMEM((C, MID), jnp.int32),
        pltpu.VMEM((C, MID // 2), jnp.int32),
        pltpu.VMEM((C, MID // 2), jnp.int32),
        pltpu.VMEM((C * OUT_DIM,), jnp.float32),
        pltpu.VMEM((C * OUT_DIM,), jnp.float32),
        pltpu.VMEM((OUT_DIM * MID,), jnp.float32),
        pltpu.VMEM((16,), jnp.float32),
        pltpu.SemaphoreType.DMA,
        pltpu.SemaphoreType.DMA,
    ],
)(_sc_readout_body)


def kernel(x, frag_idx, W1, b1, Wout, bout):
    d = x.shape[1]
    wu = jnp.concatenate([W1[0:d], W1[2 * d:3 * d]], axis=1)
    wb = W1[d:2 * d]
    third = 1.0 / 3.0
    bu = (jnp.concatenate([b1, b1]) * third)[None, :]
    bb = (b1 * third)[None, :]
    u_tab, tb_tab = _project(x, wu, wb, bu, bb)

    fi = jnp.pad(frag_idx, ((0, 0), (0, PAD_TOTAL - frag_idx.shape[1])))
    woutf = Wout.T.reshape(-1)
    boutp = jnp.pad(bout, (0, 16 - bout.shape[0]))
    out = _sc_readout(u_tab, tb_tab, fi[0], fi[1], fi[2], woutf, boutp)
    return out.reshape(PAD_TOTAL, OUT_DIM)[:frag_idx.shape[1]]


# speedup vs baseline: 2.8566x; 1.0011x over previous
import functools

import jax
import jax.numpy as jnp
from jax import lax
from jax.experimental import pallas as pl
from jax.experimental.pallas import tpu as pltpu
from jax.experimental.pallas import tpu_sc as plsc

N_ATOMS = 100000
D_FEAT = 128
MID = 32
OUT_DIM = 3
N_FRAG = 200000

NC = 2
NS = 16
NW = NC * NS
C = 320
PAD_TOTAL = -(-N_FRAG // (NW * C * 2)) * (NW * C * 2)
W_PER = PAD_TOTAL // NW
NCH = W_PER // C
NG = C // 16

ROW_BLK = 2000

_GDN = lax.GatherDimensionNumbers(
    offset_dims=(), collapsed_slice_dims=(0,), start_index_map=(0,))


def _lane_pick(vec, idx16):
    return lax.gather(vec, idx16[:, None], _GDN, (1,),
                      mode=lax.GatherScatterMode.PROMISE_IN_BOUNDS)


def _rnd_bf16_bits(x):
    b = lax.bitcast_convert_type(x, jnp.int32)
    return b + jnp.int32(0x7FFF) + ((b >> 16) & jnp.int32(1))


def _project_body(x_ref, wu_ref, wb_ref, bu_ref, bb_ref, u_ref, tb_ref):
    xb = x_ref[...]
    u = (jnp.dot(xb, wu_ref[...], preferred_element_type=jnp.float32,
                 precision=lax.Precision.DEFAULT) + bu_ref[...])
    tb = (jnp.dot(xb, wb_ref[...], preferred_element_type=jnp.float32,
                  precision=lax.Precision.DEFAULT) + bb_ref[...])
    hi_mask = jnp.int32(-65536)
    ta_b = _rnd_bf16_bits(u[:, :MID])
    tc_b = _rnd_bf16_bits(u[:, MID:])
    u_ref[...] = ((ta_b >> 16) & jnp.int32(0xFFFF)) | (tc_b & hi_mask)
    tbl_b = _rnd_bf16_bits(tb[:, :16])
    tbh_b = _rnd_bf16_bits(tb[:, 16:])
    tb_ref[...] = ((tbl_b >> 16) & jnp.int32(0xFFFF)) | (tbh_b & hi_mask)


def _project(x, wu, wb, bu, bb):
    n = x.shape[0]
    grid = n // ROW_BLK
    return pl.pallas_call(
        _project_body,
        grid=(grid,),
        in_specs=[
            pl.BlockSpec((ROW_BLK, D_FEAT), lambda i: (i, 0)),
            pl.BlockSpec((D_FEAT, 2 * MID), lambda i: (0, 0)),
            pl.BlockSpec((D_FEAT, MID), lambda i: (0, 0)),
            pl.BlockSpec((1, 2 * MID), lambda i: (0, 0)),
            pl.BlockSpec((1, MID), lambda i: (0, 0)),
        ],
        out_specs=[
            pl.BlockSpec((ROW_BLK, MID), lambda i: (i, 0)),
            pl.BlockSpec((ROW_BLK, MID // 2), lambda i: (i, 0)),
        ],
        out_shape=[
            jax.ShapeDtypeStruct((n, MID), jnp.int32),
            jax.ShapeDtypeStruct((n, MID // 2), jnp.int32),
        ],
    )(x, wu, wb, bu, bb)


def _sc_readout_body(u_hbm, tb_hbm, fi0, fi1, fi2, woutf, boutp, out_hbm,
                     idx_all, r0_a, r0_b, r2_a, r2_b, rb_a, rb_b,
                     out_a, out_b, wout_v, bout_v, sem0, sem1):
    wid = lax.axis_index("s") * NC + lax.axis_index("c")
    wbase = wid * W_PER
    sems = (sem0, sem1)
    r0s = (r0_a, r0_b)
    r2s = (r2_a, r2_b)
    rbs = (rb_a, rb_b)
    outs = (out_a, out_b)

    pltpu.sync_copy(woutf, wout_v)
    pltpu.sync_copy(boutp, bout_v)

    pltpu.sync_copy(fi0.at[pl.ds(wbase, W_PER)], idx_all.at[0])
    pltpu.sync_copy(fi1.at[pl.ds(wbase, W_PER)], idx_all.at[1])
    pltpu.sync_copy(fi2.at[pl.ds(wbase, W_PER)], idx_all.at[2])

    def stage(c, b):
        off = c * C
        pltpu.async_copy(u_hbm.at[idx_all.at[0, pl.ds(off, C)]],
                         r0s[b], sems[b])
        pltpu.async_copy(tb_hbm.at[idx_all.at[1, pl.ds(off, C)]],
                         rbs[b], sems[b])
        pltpu.async_copy(u_hbm.at[idx_all.at[2, pl.ds(off, C)]],
                         r2s[b], sems[b])

    def wait(c, b):
        off = c * C
        pltpu.make_async_copy(u_hbm.at[idx_all.at[0, pl.ds(off, C)]],
                              r0s[b], sems[b]).wait()
        pltpu.make_async_copy(tb_hbm.at[idx_all.at[1, pl.ds(off, C)]],
                              rbs[b], sems[b]).wait()
        pltpu.make_async_copy(u_hbm.at[idx_all.at[2, pl.ds(off, C)]],
                              r2s[b], sems[b]).wait()

    bv = bout_v[...]
    w_lo = [wout_v[pl.ds(j * MID, 16)] for j in range(OUT_DIM)]
    w_hi = [wout_v[pl.ds(j * MID + 16, 16)] for j in range(OUT_DIM)]
    bj = [_lane_pick(bv, jnp.full((16,), j, jnp.int32)) for j in range(OUT_DIM)]
    lane_last = lax.iota(jnp.int32, 16) == 15

    def compute(c, b):
        r0 = r0s[b]
        r2 = r2s[b]
        rb = rbs[b]
        ov = outs[b]

        @pl.loop(0, C, unroll=8)
        def _(f):
            p0l = r0[f, pl.ds(0, 16)]
            p0h = r0[f, pl.ds(16, 16)]
            p2l = r2[f, pl.ds(0, 16)]
            p2h = r2[f, pl.ds(16, 16)]
            ptb = rb[f, pl.ds(0, 16)]
            u0a = plsc.bitcast(p0l << 16, jnp.float32)
            u0c = plsc.bitcast(p0l, jnp.float32)
            u0b = plsc.bitcast(p0h << 16, jnp.float32)
            u0d = plsc.bitcast(p0h, jnp.float32)
            u2a = plsc.bitcast(p2l << 16, jnp.float32)
            u2c = plsc.bitcast(p2l, jnp.float32)
            u2b = plsc.bitcast(p2h << 16, jnp.float32)
            u2d = plsc.bitcast(p2h, jnp.float32)
            tl = plsc.bitcast(ptb << 16, jnp.float32)
            th = plsc.bitcast(ptb, jnp.float32)
            zero = jnp.float32(0.0)
            p_lo = (jnp.maximum((u0a + u2c) + tl, zero)
                    + jnp.maximum((u2a + u0c) + tl, zero))
            p_hi = (jnp.maximum((u0b + u2d) + th, zero)
                    + jnp.maximum((u2b + u0d) + th, zero))
            for j in range(OUT_DIM):
                t = p_lo * w_lo[j] + p_hi * w_hi[j]
                s = plsc.cumsum(t) + bj[j]
                plsc.store_scatter(ov, [jnp.full((16,), f * OUT_DIM + j,
                                                 jnp.int32)],
                                   s, mask=lane_last)

        pltpu.sync_copy(ov, out_hbm.at[pl.ds((wbase + c * C) * OUT_DIM,
                                             C * OUT_DIM)])

    stage(0, 0)

    @pl.loop(0, NCH // 2)
    def _(i):
        c0 = i * 2
        stage(c0 + 1, 1)
        wait(c0, 0)
        compute(c0, 0)

        @pl.when(i < NCH // 2 - 1)
        def _():
            stage(c0 + 2, 0)

        wait(c0 + 1, 1)
        compute(c0 + 1, 1)


_sc_readout = functools.partial(
    pl.kernel,
    out_type=jax.ShapeDtypeStruct((PAD_TOTAL * OUT_DIM,), jnp.float32),
    mesh=plsc.VectorSubcoreMesh(core_axis_name="c", subcore_axis_name="s",
                                num_cores=NC, num_subcores=NS),
    compiler_params=pltpu.CompilerParams(needs_layout_passes=False,
                                         use_tc_tiling_on_sc=False),
    scratch_types=[
        pltpu.VMEM((3, W_PER), jnp.int32),
        pltpu.VMEM((C, MID), jnp.int32),
        pltpu.VMEM((C, MID), jnp.int32),
        pltpu.VMEM((C, MID), jnp.int32),
        pltpu.VMEM((C, MID), jnp.int32),
        pltpu.VMEM((C, MID // 2), jnp.int32),
        pltpu.VMEM((C, MID // 2), jnp.int32),
        pltpu.VMEM((C * OUT_DIM,), jnp.float32),
        pltpu.VMEM((C * OUT_DIM,), jnp.float32),
        pltpu.VMEM((OUT_DIM * MID,), jnp.float32),
        pltpu.VMEM((16,), jnp.float32),
        pltpu.SemaphoreType.DMA,
        pltpu.SemaphoreType.DMA,
    ],
)(_sc_readout_body)


def kernel(x, frag_idx, W1, b1, Wout, bout):
    d = x.shape[1]
    wu = jnp.concatenate([W1[0:d], W1[2 * d:3 * d]], axis=1)
    wb = W1[d:2 * d]
    third = 1.0 / 3.0
    bu = (jnp.concatenate([b1, b1]) * third)[None, :]
    bb = (b1 * third)[None, :]
    u_tab, tb_tab = _project(x, wu, wb, bu, bb)

    fi = jnp.pad(frag_idx, ((0, 0), (0, PAD_TOTAL - frag_idx.shape[1])))
    woutf = Wout.T.reshape(-1)
    boutp = jnp.pad(bout, (0, 16 - bout.shape[0]))
    out = _sc_readout(u_tab, tb_tab, fi[0], fi[1], fi[2], woutf, boutp)
    return out.reshape(PAD_TOTAL, OUT_DIM)[:frag_idx.shape[1]]


# speedup vs baseline: 2.8626x; 1.0021x over previous
import functools

import jax
import jax.numpy as jnp
from jax import lax
from jax.experimental import pallas as pl
from jax.experimental.pallas import tpu as pltpu
from jax.experimental.pallas import tpu_sc as plsc

N_ATOMS = 100000
D_FEAT = 128
MID = 32
OUT_DIM = 3
N_FRAG = 200000

NC = 2
NS = 16
NW = NC * NS
C = 320
PAD_TOTAL = -(-N_FRAG // (NW * C * 2)) * (NW * C * 2)
W_PER = PAD_TOTAL // NW
NCH = W_PER // C

ROW_BLK = 2000

_GDN = lax.GatherDimensionNumbers(
    offset_dims=(), collapsed_slice_dims=(0,), start_index_map=(0,))


def _lane_pick(vec, idx16):
    return lax.gather(vec, idx16[:, None], _GDN, (1,),
                      mode=lax.GatherScatterMode.PROMISE_IN_BOUNDS)


def _rnd_bf16_bits(x):
    b = lax.bitcast_convert_type(x, jnp.int32)
    return b + jnp.int32(0x7FFF) + ((b >> 16) & jnp.int32(1))


def _project_body(x_ref, wu_ref, wb_ref, bu_ref, bb_ref, u_ref, tb_ref):
    xb = x_ref[...]
    u = (jnp.dot(xb, wu_ref[...], preferred_element_type=jnp.float32,
                 precision=lax.Precision.DEFAULT) + bu_ref[...])
    tb = (jnp.dot(xb, wb_ref[...], preferred_element_type=jnp.float32,
                  precision=lax.Precision.DEFAULT) + bb_ref[...])
    hi_mask = jnp.int32(-65536)
    ta_b = _rnd_bf16_bits(u[:, :MID])
    tc_b = _rnd_bf16_bits(u[:, MID:])
    u_ref[...] = ((ta_b >> 16) & jnp.int32(0xFFFF)) | (tc_b & hi_mask)
    tbl_b = _rnd_bf16_bits(tb[:, :16])
    tbh_b = _rnd_bf16_bits(tb[:, 16:])
    tb_ref[...] = ((tbl_b >> 16) & jnp.int32(0xFFFF)) | (tbh_b & hi_mask)


def _project(x, wu, wb, bu, bb):
    n = x.shape[0]
    grid = n // ROW_BLK
    return pl.pallas_call(
        _project_body,
        grid=(grid,),
        in_specs=[
            pl.BlockSpec((ROW_BLK, D_FEAT), lambda i: (i, 0)),
            pl.BlockSpec((D_FEAT, 2 * MID), lambda i: (0, 0)),
            pl.BlockSpec((D_FEAT, MID), lambda i: (0, 0)),
            pl.BlockSpec((1, 2 * MID), lambda i: (0, 0)),
            pl.BlockSpec((1, MID), lambda i: (0, 0)),
        ],
        out_specs=[
            pl.BlockSpec((ROW_BLK, MID), lambda i: (i, 0)),
            pl.BlockSpec((ROW_BLK, MID // 2), lambda i: (i, 0)),
        ],
        out_shape=[
            jax.ShapeDtypeStruct((n, MID), jnp.int32),
            jax.ShapeDtypeStruct((n, MID // 2), jnp.int32),
        ],
    )(x, wu, wb, bu, bb)


def _sc_readout_body(u_hbm, tb_hbm, fi0, fi1, fi2, woutf, boutp, out_hbm,
                     idx_all, r0_a, r0_b, r2_a, r2_b, rb_a, rb_b,
                     out_a, out_b, wout_v, bout_v, sem0, sem1):
    wid = lax.axis_index("s") * NC + lax.axis_index("c")
    wbase = wid * W_PER
    sems = (sem0, sem1)
    r0s = (r0_a, r0_b)
    r2s = (r2_a, r2_b)
    rbs = (rb_a, rb_b)
    outs = (out_a, out_b)

    pltpu.sync_copy(woutf, wout_v)
    pltpu.sync_copy(boutp, bout_v)

    pltpu.sync_copy(fi0.at[pl.ds(wbase, W_PER)], idx_all.at[0])
    pltpu.sync_copy(fi1.at[pl.ds(wbase, W_PER)], idx_all.at[1])
    pltpu.sync_copy(fi2.at[pl.ds(wbase, W_PER)], idx_all.at[2])

    def stage(c, b):
        off = c * C
        pltpu.async_copy(u_hbm.at[idx_all.at[0, pl.ds(off, C)]],
                         r0s[b], sems[b])
        pltpu.async_copy(tb_hbm.at[idx_all.at[1, pl.ds(off, C)]],
                         rbs[b], sems[b])
        pltpu.async_copy(u_hbm.at[idx_all.at[2, pl.ds(off, C)]],
                         r2s[b], sems[b])

    def wait(c, b):
        off = c * C
        pltpu.make_async_copy(u_hbm.at[idx_all.at[0, pl.ds(off, C)]],
                              r0s[b], sems[b]).wait()
        pltpu.make_async_copy(tb_hbm.at[idx_all.at[1, pl.ds(off, C)]],
                              rbs[b], sems[b]).wait()
        pltpu.make_async_copy(u_hbm.at[idx_all.at[2, pl.ds(off, C)]],
                              r2s[b], sems[b]).wait()

    bv = bout_v[...]
    w_lo = [wout_v[pl.ds(j * MID, 16)] for j in range(OUT_DIM)]
    w_hi = [wout_v[pl.ds(j * MID + 16, 16)] for j in range(OUT_DIM)]
    bj = [_lane_pick(bv, jnp.full((16,), j, jnp.int32)) for j in range(OUT_DIM)]
    lane_last = lax.iota(jnp.int32, 16) == 15

    def compute(c, b):
        r0 = r0s[b]
        r2 = r2s[b]
        rb = rbs[b]
        ov = outs[b]

        @pl.loop(0, C, unroll=8)
        def _(f):
            p0l = r0[f, pl.ds(0, 16)]
            p0h = r0[f, pl.ds(16, 16)]
            p2l = r2[f, pl.ds(0, 16)]
            p2h = r2[f, pl.ds(16, 16)]
            ptb = rb[f, pl.ds(0, 16)]
            u0a = plsc.bitcast(p0l << 16, jnp.float32)
            u0c = plsc.bitcast(p0l, jnp.float32)
            u0b = plsc.bitcast(p0h << 16, jnp.float32)
            u0d = plsc.bitcast(p0h, jnp.float32)
            u2a = plsc.bitcast(p2l << 16, jnp.float32)
            u2c = plsc.bitcast(p2l, jnp.float32)
            u2b = plsc.bitcast(p2h << 16, jnp.float32)
            u2d = plsc.bitcast(p2h, jnp.float32)
            tl = plsc.bitcast(ptb << 16, jnp.float32)
            th = plsc.bitcast(ptb, jnp.float32)
            zero = jnp.float32(0.0)
            p_lo = (jnp.maximum((u0a + u2c) + tl, zero)
                    + jnp.maximum((u2a + u0c) + tl, zero))
            p_hi = (jnp.maximum((u0b + u2d) + th, zero)
                    + jnp.maximum((u2b + u0d) + th, zero))
            for j in range(OUT_DIM):
                t = p_lo * w_lo[j] + p_hi * w_hi[j]
                s = plsc.cumsum(t) + bj[j]
                plsc.store_scatter(ov, [jnp.full((16,), f * OUT_DIM + j,
                                                 jnp.int32)],
                                   s, mask=lane_last)

        pltpu.sync_copy(ov, out_hbm.at[pl.ds((wbase + c * C) * OUT_DIM,
                                             C * OUT_DIM)])

    stage(0, 0)

    @pl.loop(0, NCH // 2)
    def _(i):
        c0 = i * 2
        stage(c0 + 1, 1)
        wait(c0, 0)
        compute(c0, 0)

        @pl.when(i < NCH // 2 - 1)
        def _():
            stage(c0 + 2, 0)

        wait(c0 + 1, 1)
        compute(c0 + 1, 1)


_sc_readout = functools.partial(
    pl.kernel,
    out_type=jax.ShapeDtypeStruct((PAD_TOTAL * OUT_DIM,), jnp.float32),
    mesh=plsc.VectorSubcoreMesh(core_axis_name="c", subcore_axis_name="s",
                                num_cores=NC, num_subcores=NS),
    compiler_params=pltpu.CompilerParams(needs_layout_passes=False,
                                         use_tc_tiling_on_sc=False),
    scratch_types=[
        pltpu.VMEM((3, W_PER), jnp.int32),
        pltpu.VMEM((C, MID), jnp.int32),
        pltpu.VMEM((C, MID), jnp.int32),
        pltpu.VMEM((C, MID), jnp.int32),
        pltpu.VMEM((C, MID), jnp.int32),
        pltpu.VMEM((C, MID // 2), jnp.int32),
        pltpu.VMEM((C, MID // 2), jnp.int32),
        pltpu.VMEM((C * OUT_DIM,), jnp.float32),
        pltpu.VMEM((C * OUT_DIM,), jnp.float32),
        pltpu.VMEM((OUT_DIM * MID,), jnp.float32),
        pltpu.VMEM((16,), jnp.float32),
        pltpu.SemaphoreType.DMA,
        pltpu.SemaphoreType.DMA,
    ],
)(_sc_readout_body)


def kernel(x, frag_idx, W1, b1, Wout, bout):
    d = x.shape[1]
    wu = jnp.concatenate([W1[0:d], W1[2 * d:3 * d]], axis=1)
    wb = W1[d:2 * d]
    third = 1.0 / 3.0
    bu = (jnp.concatenate([b1, b1]) * third)[None, :]
    bb = (b1 * third)[None, :]
    u_tab, tb_tab = _project(x, wu, wb, bu, bb)

    fi = jnp.pad(frag_idx, ((0, 0), (0, PAD_TOTAL - frag_idx.shape[1])))
    woutf = Wout.T.reshape(-1)
    boutp = jnp.pad(bout, (0, 16 - bout.shape[0]))
    out = _sc_readout(u_tab, tb_tab, fi[0], fi[1], fi[2], woutf, boutp)
    return out.reshape(PAD_TOTAL, OUT_DIM)[:frag_idx.shape[1]]
